# Initial kernel scaffold; baseline (speedup 1.0000x reference)
import functools
import numpy as np
import jax
import jax.numpy as jnp
from jax import lax
from jax.experimental import pallas as pl
from jax.experimental.pallas import tpu as pltpu
from jax.experimental.pallas import tpu_sc as plsc

B = 16384
SEQ = 50
D = 64
NC, NS, L = 2, 16, 16
NW = NC * NS
SEQS_PER_W = B // NW
S_CHUNK = 8
TOK_CHUNK = S_CHUNK * SEQ
CHUNKS = SEQS_PER_W // S_CHUNK


def _make_pe_const():
    position = np.arange(0, SEQ, dtype=np.float32)[:, None]
    div_term = np.exp(
        np.arange(0, D, 2, dtype=np.float32) * (-np.log(10000.0) / D))
    pe = np.zeros((SEQ, D), dtype=np.float32)
    pe[:, 0::2] = np.sin(position * div_term)
    pe[:, 1::2] = np.cos(position * div_term)
    return jnp.asarray(pe)


_MESH = plsc.VectorSubcoreMesh(core_axis_name="c", subcore_axis_name="s")


@functools.partial(
    pl.kernel,
    mesh=_MESH,
    out_type=jax.ShapeDtypeStruct((B * SEQ, D), jnp.float32),
    scratch_types=[
        pltpu.VMEM((TOK_CHUNK,), jnp.int32),
        pltpu.VMEM((TOK_CHUNK, D), jnp.float32),
        pltpu.VMEM((SEQ, D), jnp.float32),
        pltpu.SemaphoreType.DMA,
    ],
)
def _embed_pe(tok_hbm, table_hbm, pe_hbm, out_hbm, idx_v, rows_v, pe_v, sem):
    wid = lax.axis_index("s") * NC + lax.axis_index("c")
    base = wid * SEQS_PER_W * SEQ
    pltpu.sync_copy(pe_hbm, pe_v)

    def chunk_body(g, carry):
        off = base + g * TOK_CHUNK
        pltpu.sync_copy(tok_hbm.at[pl.ds(off, TOK_CHUNK)], idx_v)
        pltpu.async_copy(table_hbm.at[idx_v], rows_v, sem).wait()

        def pos_body(p, c2):
            pe_regs = [pe_v[p, pl.ds(c * L, L)] for c in range(D // L)]

            def seq_body(s, c3):
                r = s * SEQ + p
                for c in range(D // L):
                    rows_v[r, pl.ds(c * L, L)] = (
                        rows_v[r, pl.ds(c * L, L)] + pe_regs[c])
                return c3

            return lax.fori_loop(0, S_CHUNK, seq_body, c2)

        lax.fori_loop(0, SEQ, pos_body, 0)
        pltpu.sync_copy(rows_v, out_hbm.at[pl.ds(off, TOK_CHUNK)])
        return carry

    lax.fori_loop(0, CHUNKS, chunk_body, 0)


def kernel(tokens, table):
    pe = _make_pe_const()
    tok_flat = tokens.reshape(-1).astype(jnp.int32)
    out = _embed_pe(tok_flat, table, pe)
    return out.reshape(B, SEQ, D)


# speedup vs baseline: 1.6888x; 1.6888x over previous
import functools
import numpy as np
import jax
import jax.numpy as jnp
from jax import lax
from jax.experimental import pallas as pl
from jax.experimental.pallas import tpu as pltpu
from jax.experimental.pallas import tpu_sc as plsc

B = 16384
SEQ = 50
D = 64
NC, NS, L = 2, 16, 16
NW = NC * NS
SEQS_PER_W = B // NW
S_CHUNK = 8
TOK_CHUNK = S_CHUNK * SEQ
CHUNKS = SEQS_PER_W // S_CHUNK


def _make_pe_const():
    position = np.arange(0, SEQ, dtype=np.float32)[:, None]
    div_term = np.exp(
        np.arange(0, D, 2, dtype=np.float32) * (-np.log(10000.0) / D))
    pe = np.zeros((SEQ, D), dtype=np.float32)
    pe[:, 0::2] = np.sin(position * div_term)
    pe[:, 1::2] = np.cos(position * div_term)
    return jnp.asarray(pe)


_MESH = plsc.VectorSubcoreMesh(core_axis_name="c", subcore_axis_name="s")


@functools.partial(
    pl.kernel,
    mesh=_MESH,
    compiler_params=pltpu.CompilerParams(use_tc_tiling_on_sc=False),
    out_type=jax.ShapeDtypeStruct((B * SEQ, D), jnp.float32),
    scratch_types=[
        pltpu.VMEM((TOK_CHUNK,), jnp.int32),
        pltpu.VMEM((TOK_CHUNK, D), jnp.float32),
        pltpu.VMEM((SEQ, D), jnp.float32),
        pltpu.SemaphoreType.DMA,
    ],
)
def _embed_pe(tok_hbm, table_hbm, pe_hbm, out_hbm, idx_v, rows_v, pe_v, sem):
    wid = lax.axis_index("s") * NC + lax.axis_index("c")
    base = wid * SEQS_PER_W * SEQ
    pltpu.sync_copy(pe_hbm, pe_v)

    def chunk_body(g, carry):
        off = base + g * TOK_CHUNK
        pltpu.sync_copy(tok_hbm.at[pl.ds(off, TOK_CHUNK)], idx_v)
        pltpu.async_copy(table_hbm.at[idx_v], rows_v, sem).wait()

        def pos_body(p, c2):
            pe_regs = [pe_v[p, pl.ds(c * L, L)] for c in range(D // L)]

            def seq_body(s, c3):
                r = s * SEQ + p
                for c in range(D // L):
                    rows_v[r, pl.ds(c * L, L)] = (
                        rows_v[r, pl.ds(c * L, L)] + pe_regs[c])
                return c3

            return lax.fori_loop(0, S_CHUNK, seq_body, c2)

        lax.fori_loop(0, SEQ, pos_body, 0)
        pltpu.sync_copy(rows_v, out_hbm.at[pl.ds(off, TOK_CHUNK)])
        return carry

    lax.fori_loop(0, CHUNKS, chunk_body, 0)


def kernel(tokens, table):
    pe = _make_pe_const()
    tok_flat = tokens.reshape(-1).astype(jnp.int32)
    out = _embed_pe(tok_flat, table, pe)
    return out.reshape(B, SEQ, D)


# speedup vs baseline: 1.8721x; 1.1086x over previous
import functools
import numpy as np
import jax
import jax.numpy as jnp
from jax import lax
from jax.experimental import pallas as pl
from jax.experimental.pallas import tpu as pltpu
from jax.experimental.pallas import tpu_sc as plsc

B = 16384
SEQ = 50
D = 64
NC, NS, L = 2, 16, 16
NW = NC * NS
SEQS_PER_W = B // NW
S_CHUNK = 8
TOK_CHUNK = S_CHUNK * SEQ
CHUNKS = SEQS_PER_W // S_CHUNK
NBUF = 4
NVR = D // L


def _make_pe_const():
    position = np.arange(0, SEQ, dtype=np.float32)[:, None]
    div_term = np.exp(
        np.arange(0, D, 2, dtype=np.float32) * (-np.log(10000.0) / D))
    pe = np.zeros((SEQ, D), dtype=np.float32)
    pe[:, 0::2] = np.sin(position * div_term)
    pe[:, 1::2] = np.cos(position * div_term)
    return jnp.asarray(pe)


_MESH = plsc.VectorSubcoreMesh(core_axis_name="c", subcore_axis_name="s")


@functools.partial(
    pl.kernel,
    mesh=_MESH,
    compiler_params=pltpu.CompilerParams(use_tc_tiling_on_sc=False),
    out_type=jax.ShapeDtypeStruct((B * SEQ, D), jnp.float32),
    scratch_types=[
        pltpu.VMEM((NBUF, TOK_CHUNK), jnp.int32),
        pltpu.VMEM((NBUF, TOK_CHUNK, D), jnp.float32),
        pltpu.VMEM((SEQ, D), jnp.float32),
    ]
    + [pltpu.SemaphoreType.DMA] * (2 * NBUF),
)
def _embed_pe(tok_hbm, table_hbm, pe_hbm, out_hbm, idx_v, rows_v, pe_v, *sems):
    gsem = sems[:NBUF]
    ssem = sems[NBUF:]
    wid = lax.axis_index("s") * NC + lax.axis_index("c")
    base = wid * SEQS_PER_W * SEQ
    pltpu.sync_copy(pe_hbm, pe_v)

    def start_gather(g, b):
        off = base + g * TOK_CHUNK
        pltpu.sync_copy(tok_hbm.at[pl.ds(off, TOK_CHUNK)], idx_v.at[b])
        pltpu.async_copy(table_hbm.at[idx_v.at[b]], rows_v.at[b], gsem[b])

    def wait_gather(b):
        pltpu.make_async_copy(
            table_hbm.at[idx_v.at[b]], rows_v.at[b], gsem[b]).wait()

    def wait_scatter(b):
        pltpu.make_async_copy(
            rows_v.at[b], out_hbm.at[pl.ds(0, TOK_CHUNK)], ssem[b]).wait()

    def add_pe(b):
        rows_b = rows_v.at[b]

        def pos_body(p, carry):
            pe_regs = [pe_v[p, pl.ds(c * L, L)] for c in range(NVR)]
            for s in range(S_CHUNK):
                r = s * SEQ + p
                for c in range(NVR):
                    rows_b[r, pl.ds(c * L, L)] = (
                        rows_b[r, pl.ds(c * L, L)] + pe_regs[c])
            return carry

        lax.fori_loop(0, SEQ, pos_body, 0)

    start_gather(0, 0)
    start_gather(1, 1)

    def ring_body(i, carry):
        for b in range(NBUF):
            g = i * NBUF + b
            bn = (b + 2) % NBUF

            @pl.when(g + 2 < CHUNKS)
            def _issue_next():
                @pl.when(g + 2 - NBUF >= 0)
                def _drain_prev_scatter():
                    wait_scatter(bn)

                start_gather(g + 2, bn)

            wait_gather(b)
            add_pe(b)
            pltpu.async_copy(
                rows_v.at[b],
                out_hbm.at[pl.ds(base + g * TOK_CHUNK, TOK_CHUNK)],
                ssem[b])
        return carry

    lax.fori_loop(0, CHUNKS // NBUF, ring_body, 0)
    for b in range(NBUF):
        wait_scatter(b)


def kernel(tokens, table):
    pe = _make_pe_const()
    tok_flat = tokens.reshape(-1).astype(jnp.int32)
    out = _embed_pe(tok_flat, table, pe)
    return out.reshape(B, SEQ, D)
